# Initial kernel scaffold; baseline (speedup 1.0000x reference)
import functools

import jax
import jax.numpy as jnp
from jax.experimental import pallas as pl
from jax.experimental.pallas import tpu as pltpu

B, S, D = 4, 8192, 1024
H = D // 2
K = 40
BLK = 1024
NBLK = (B * S) // BLK
RPB = S // BLK


def _scorer_body(x_ref, w1_ref, b1_ref, g_ref, be_ref, w2_ref, b2_ref, out_ref):
    x = x_ref[0]
    h = jnp.dot(x, w1_ref[...], preferred_element_type=jnp.float32,
                precision=jax.lax.Precision.HIGHEST) + b1_ref[...]
    mu = jnp.mean(h, axis=1, keepdims=True)
    var = jnp.mean((h - mu) ** 2, axis=1, keepdims=True)
    hn = (h - mu) / jnp.sqrt(var + 1e-5) * g_ref[...] + be_ref[...]
    ge = jax.nn.gelu(hn, approximate=False)
    logit = jnp.dot(ge, w2_ref[...], preferred_element_type=jnp.float32,
                    precision=jax.lax.Precision.HIGHEST) + b2_ref[...]
    out_ref[0] = jax.nn.sigmoid(logit)


def _topk_body(scores_ref, feat_ref, idx_ref, tok_ref, sem):
    flat = (jax.lax.broadcasted_iota(jnp.int32, (RPB, BLK), 0) * BLK
            + jax.lax.broadcasted_iota(jnp.int32, (RPB, BLK), 1))
    lane_j = jax.lax.broadcasted_iota(jnp.int32, (B, K), 1)
    row_b = jax.lax.broadcasted_iota(jnp.int32, (B, K), 0)
    idx_all = jnp.zeros((B, K), jnp.int32)
    copies = []
    for b in range(B):
        work = scores_ref[b]
        for j in range(K):
            m = jnp.max(work)
            idx = jnp.min(jnp.where(work == m, flat, jnp.int32(S)))
            work = jnp.where(flat == idx, jnp.float32(-1.0), work)
            idx_all = jnp.where((row_b == b) & (lane_j == j), idx, idx_all)
            c = pltpu.make_async_copy(feat_ref.at[b, idx], tok_ref.at[b, j], sem)
            c.start()
            copies.append(c)
    idx_ref[...] = idx_all
    for c in copies:
        c.wait()


@jax.jit
def kernel(features, W1, b1, gamma, beta, W2, b2):
    xf = features.reshape(NBLK, BLK, D)
    scores = pl.pallas_call(
        _scorer_body,
        grid=(NBLK,),
        in_specs=[
            pl.BlockSpec((1, BLK, D), lambda s: (s, 0, 0)),
            pl.BlockSpec((D, H), lambda s: (0, 0)),
            pl.BlockSpec((1, H), lambda s: (0, 0)),
            pl.BlockSpec((1, H), lambda s: (0, 0)),
            pl.BlockSpec((1, H), lambda s: (0, 0)),
            pl.BlockSpec((H, 1), lambda s: (0, 0)),
            pl.BlockSpec((1, 1), lambda s: (0, 0)),
        ],
        out_specs=pl.BlockSpec((1, BLK, 1), lambda s: (s, 0, 0)),
        out_shape=jax.ShapeDtypeStruct((NBLK, BLK, 1), jnp.float32),
    )(xf, W1, b1.reshape(1, H), gamma.reshape(1, H), beta.reshape(1, H),
      W2, b2.reshape(1, 1))

    scores_bt = scores.reshape(B, RPB, BLK)
    indices, tokens = pl.pallas_call(
        _topk_body,
        in_specs=[
            pl.BlockSpec(memory_space=pltpu.VMEM),
            pl.BlockSpec(memory_space=pltpu.ANY),
        ],
        out_specs=[
            pl.BlockSpec(memory_space=pltpu.VMEM),
            pl.BlockSpec(memory_space=pltpu.ANY),
        ],
        out_shape=[
            jax.ShapeDtypeStruct((B, K), jnp.int32),
            jax.ShapeDtypeStruct((B, K, D), jnp.float32),
        ],
        scratch_shapes=[pltpu.SemaphoreType.DMA],
    )(scores_bt, features)
    return (tokens, indices)


# speedup vs baseline: 1.7872x; 1.7872x over previous
import functools

import jax
import jax.numpy as jnp
from jax.experimental import pallas as pl
from jax.experimental.pallas import tpu as pltpu

B, S, D = 4, 8192, 1024
H = D // 2
K = 40
BLK = 1024
NBLK = (B * S) // BLK
RPB = S // BLK


def _scorer_body(x_ref, w1_ref, b1_ref, g_ref, be_ref, w2_ref, b2_ref, out_ref):
    x = x_ref[0]
    h = jnp.dot(x, w1_ref[...], preferred_element_type=jnp.float32,
                precision=jax.lax.Precision.DEFAULT) + b1_ref[...]
    mu = jnp.mean(h, axis=1, keepdims=True)
    var = jnp.mean((h - mu) ** 2, axis=1, keepdims=True)
    hn = (h - mu) / jnp.sqrt(var + 1e-5) * g_ref[...] + be_ref[...]
    ge = hn * 0.5 * (1.0 + jax.lax.erf(hn * jnp.float32(0.7071067811865476)))
    logit = jnp.dot(ge, w2_ref[...], preferred_element_type=jnp.float32,
                    precision=jax.lax.Precision.DEFAULT) + b2_ref[...]
    out_ref[0] = jax.nn.sigmoid(logit)


def _topk_body(scores_ref, feat_ref, idx_ref, tok_ref, sem):
    flat = (jax.lax.broadcasted_iota(jnp.int32, (RPB, BLK), 0) * BLK
            + jax.lax.broadcasted_iota(jnp.int32, (RPB, BLK), 1))
    lane_j = jax.lax.broadcasted_iota(jnp.int32, (B, K), 1)
    row_b = jax.lax.broadcasted_iota(jnp.int32, (B, K), 0)
    idx_all = jnp.zeros((B, K), jnp.int32)
    copies = []
    for b in range(B):
        work = scores_ref[b]
        for j in range(K):
            m = jnp.max(work)
            idx = jnp.min(jnp.where(work == m, flat, jnp.int32(S)))
            work = jnp.where(flat == idx, jnp.float32(-1.0), work)
            idx_all = jnp.where((row_b == b) & (lane_j == j), idx, idx_all)
            c = pltpu.make_async_copy(feat_ref.at[b, idx], tok_ref.at[b, j], sem)
            c.start()
            copies.append(c)
    idx_ref[...] = idx_all
    for c in copies:
        c.wait()


@jax.jit
def kernel(features, W1, b1, gamma, beta, W2, b2):
    xf = features.reshape(NBLK, BLK, D)
    scores = pl.pallas_call(
        _scorer_body,
        grid=(NBLK,),
        in_specs=[
            pl.BlockSpec((1, BLK, D), lambda s: (s, 0, 0)),
            pl.BlockSpec((D, H), lambda s: (0, 0)),
            pl.BlockSpec((1, H), lambda s: (0, 0)),
            pl.BlockSpec((1, H), lambda s: (0, 0)),
            pl.BlockSpec((1, H), lambda s: (0, 0)),
            pl.BlockSpec((H, 1), lambda s: (0, 0)),
            pl.BlockSpec((1, 1), lambda s: (0, 0)),
        ],
        out_specs=pl.BlockSpec((1, BLK, 1), lambda s: (s, 0, 0)),
        out_shape=jax.ShapeDtypeStruct((NBLK, BLK, 1), jnp.float32),
    )(xf, W1, b1.reshape(1, H), gamma.reshape(1, H), beta.reshape(1, H),
      W2, b2.reshape(1, 1))

    scores_bt = scores.reshape(B, RPB, BLK)
    indices, tokens = pl.pallas_call(
        _topk_body,
        in_specs=[
            pl.BlockSpec(memory_space=pltpu.MemorySpace.VMEM),
            pl.BlockSpec(memory_space=pl.ANY),
        ],
        out_specs=[
            pl.BlockSpec(memory_space=pltpu.MemorySpace.VMEM),
            pl.BlockSpec(memory_space=pl.ANY),
        ],
        out_shape=[
            jax.ShapeDtypeStruct((B, K), jnp.int32),
            jax.ShapeDtypeStruct((B, K, D), jnp.float32),
        ],
        scratch_shapes=[pltpu.SemaphoreType.DMA],
    )(scores_bt, features)
    return (tokens, indices)


# speedup vs baseline: 1.9632x; 1.0985x over previous
import functools

import jax
import jax.numpy as jnp
from jax.experimental import pallas as pl
from jax.experimental.pallas import tpu as pltpu
from jax.experimental.pallas import tpu_sc as plsc

B, S, D = 4, 8192, 1024
H = D // 2
K = 40
BLK = 1024
NBLK = (B * S) // BLK
CH = 1024
NCH = CH // 16
WPB = S // CH
MCH = (WPB * K) // 16
BIGI = 1 << 30


def _scorer_body(x_ref, w1_ref, b1_ref, g_ref, be_ref, w2_ref, b2_ref, out_ref):
    x = x_ref[0]
    h = jnp.dot(x, w1_ref[...], preferred_element_type=jnp.float32,
                precision=jax.lax.Precision.DEFAULT) + b1_ref[...]
    mu = jnp.mean(h, axis=1, keepdims=True)
    var = jnp.mean((h - mu) ** 2, axis=1, keepdims=True)
    hn = (h - mu) / jnp.sqrt(var + 1e-5) * g_ref[...] + be_ref[...]
    ge = hn * 0.5 * (1.0 + jax.lax.erf(hn * jnp.float32(0.7071067811865476)))
    logit = jnp.dot(ge, w2_ref[...], preferred_element_type=jnp.float32,
                    precision=jax.lax.Precision.DEFAULT) + b2_ref[...]
    out_ref[0] = jax.nn.sigmoid(logit)


def _sc_topk_body(scores_hbm, feat_hbm, idx_hbm, tok_hbm,
                  sv, lv, li, cand_v, cand_i, mv, mi, gi_v, oi_v, rows_v,
                  sem):
    cid = jax.lax.axis_index("c")
    sid = jax.lax.axis_index("s")
    b = cid * 2 + sid // WPB
    w = sid % WPB
    base = b * S + w * CH

    lane = jax.lax.iota(jnp.int32, 16)
    ones = jnp.full((16,), 1, jnp.int32)
    lane0 = lane == 0

    pltpu.sync_copy(scores_hbm.at[pl.ds(base, CH)], sv)

    def select_local(j, carry):
        def scan_chunk(i, c):
            cm, ci = c
            v = sv[pl.ds(i * 16, 16)]
            ii = lane + i * 16
            better = v > cm
            return (jnp.where(better, v, cm), jnp.where(better, ii, ci))
        cm, ci = jax.lax.fori_loop(
            0, NCH, scan_chunk,
            (jnp.full((16,), -2.0, jnp.float32), jnp.zeros((16,), jnp.int32)))
        m = jnp.max(cm)
        loc = jnp.min(jnp.where(cm == m, ci, BIGI))
        plsc.store_scatter(sv, [loc * ones],
                           jnp.full((16,), -1.0, jnp.float32), mask=lane0)
        plsc.store_scatter(lv, [j * ones],
                           jnp.full((16,), 1.0, jnp.float32) * m, mask=lane0)
        plsc.store_scatter(li, [j * ones], (base + loc) * ones, mask=lane0)
        return carry
    jax.lax.fori_loop(0, K, select_local, 0)

    pltpu.sync_copy(lv, cand_v.at[pl.ds(sid * K, K)])
    pltpu.sync_copy(li, cand_i.at[pl.ds(sid * K, K)])
    plsc.subcore_barrier()

    @pl.when(w == 0)
    def _merge():
        pltpu.sync_copy(cand_v.at[pl.ds(sid * K, WPB * K)], mv)
        pltpu.sync_copy(cand_i.at[pl.ds(sid * K, WPB * K)], mi)

        def select_glob(j, carry):
            def scan_chunk(i, c):
                cv, cidx, cpos = c
                v = mv[pl.ds(i * 16, 16)]
                ii = mi[pl.ds(i * 16, 16)]
                pos = lane + i * 16
                better = (v > cv) | ((v == cv) & (ii < cidx))
                return (jnp.where(better, v, cv),
                        jnp.where(better, ii, cidx),
                        jnp.where(better, pos, cpos))
            cv, cidx, cpos = jax.lax.fori_loop(
                0, MCH, scan_chunk,
                (jnp.full((16,), -2.0, jnp.float32),
                 jnp.full((16,), BIGI, jnp.int32),
                 jnp.zeros((16,), jnp.int32)))
            m = jnp.max(cv)
            hit = cv == m
            g = jnp.min(jnp.where(hit, cidx, BIGI))
            p = jnp.min(jnp.where(hit & (cidx == g), cpos, BIGI))
            plsc.store_scatter(mv, [p * ones],
                               jnp.full((16,), -1.0, jnp.float32), mask=lane0)
            plsc.store_scatter(gi_v, [j * ones], g * ones, mask=lane0)
            plsc.store_scatter(oi_v, [j * ones], (g - b * S) * ones,
                               mask=lane0)
            return carry
        jax.lax.fori_loop(0, K, select_glob, 0)

        pltpu.sync_copy(oi_v, idx_hbm.at[b])
        pltpu.async_copy(feat_hbm.at[gi_v], rows_v, sem).wait()
        pltpu.sync_copy(rows_v, tok_hbm.at[b])


@jax.jit
def kernel(features, W1, b1, gamma, beta, W2, b2):
    xf = features.reshape(NBLK, BLK, D)
    scores = pl.pallas_call(
        _scorer_body,
        grid=(NBLK,),
        in_specs=[
            pl.BlockSpec((1, BLK, D), lambda s: (s, 0, 0)),
            pl.BlockSpec((D, H), lambda s: (0, 0)),
            pl.BlockSpec((1, H), lambda s: (0, 0)),
            pl.BlockSpec((1, H), lambda s: (0, 0)),
            pl.BlockSpec((1, H), lambda s: (0, 0)),
            pl.BlockSpec((H, 1), lambda s: (0, 0)),
            pl.BlockSpec((1, 1), lambda s: (0, 0)),
        ],
        out_specs=pl.BlockSpec((1, BLK, 1), lambda s: (s, 0, 0)),
        out_shape=jax.ShapeDtypeStruct((NBLK, BLK, 1), jnp.float32),
    )(xf, W1, b1.reshape(1, H), gamma.reshape(1, H), beta.reshape(1, H),
      W2, b2.reshape(1, 1))

    sc_topk = pl.kernel(
        _sc_topk_body,
        out_type=[jax.ShapeDtypeStruct((B, K), jnp.int32),
                  jax.ShapeDtypeStruct((B, K, D), jnp.float32)],
        mesh=plsc.VectorSubcoreMesh(core_axis_name="c", subcore_axis_name="s"),
        compiler_params=pltpu.CompilerParams(needs_layout_passes=False),
        scratch_types=[
            pltpu.VMEM((CH,), jnp.float32),
            pltpu.VMEM((K,), jnp.float32),
            pltpu.VMEM((K,), jnp.int32),
            pltpu.VMEM_SHARED((16 * K,), jnp.float32),
            pltpu.VMEM_SHARED((16 * K,), jnp.int32),
            pltpu.VMEM((WPB * K,), jnp.float32),
            pltpu.VMEM((WPB * K,), jnp.int32),
            pltpu.VMEM((K,), jnp.int32),
            pltpu.VMEM((K,), jnp.int32),
            pltpu.VMEM((K, D), jnp.float32),
            pltpu.SemaphoreType.DMA,
        ],
    )
    indices, tokens = sc_topk(scores.reshape(B * S), features.reshape(B * S, D))
    return (tokens, indices)
